# baseline (device time: 407714 ns/iter reference)
import jax
import jax.numpy as jnp
from jax import lax
from jax.experimental import pallas as pl
from jax.experimental.pallas import tpu as pltpu

N_DEV = 16


def kernel(x, router_W, route_idx, expert_W, shared_W):
    n_tok, d = x.shape
    n_exp = router_W.shape[1]
    e_loc = expert_W.shape[0]
    h = shared_W.shape[1]
    ACC0 = d
    ACC1 = d + h
    RC = d + h
    PKT = d + h + 128

    def body(x_ref, rw_ref, idx_ref, ew_ref, sw_ref, out_ref,
             pkt, send_sems, recv_sems, credit):
        my = lax.axis_index("i")
        left = lax.rem(my + N_DEV - 1, N_DEV)
        right = lax.rem(my + 1, N_DEV)

        barrier = pltpu.get_barrier_semaphore()
        for nbr in (left, right):
            pl.semaphore_signal(
                barrier, inc=1, device_id=(nbr,),
                device_id_type=pl.DeviceIdType.MESH,
            )
        pl.semaphore_wait(barrier, 2)

        xv = x_ref[...]
        scores = jnp.dot(xv, rw_ref[...], preferred_element_type=jnp.float32)
        m = jnp.max(scores, axis=1, keepdims=True)
        p = jnp.exp(scores - m)
        p = p / jnp.sum(p, axis=1, keepdims=True)
        route = idx_ref[...]
        onehot = lax.broadcasted_iota(jnp.int32, (n_tok, n_exp), 1) == route
        coeff = jnp.sum(jnp.where(onehot, p, 0.0), axis=1, keepdims=True)
        pkt[0, :, 0:d] = xv * coeff
        pkt[0, :, ACC0:ACC1] = jnp.dot(
            xv, sw_ref[...], preferred_element_type=jnp.float32)
        pkt[0, :, RC:RC + 128] = jnp.broadcast_to(
            route.astype(jnp.float32), (n_tok, 128))

        ew = ew_ref[...]
        my_base = (my * e_loc).astype(jnp.float32)

        def process(slot):
            xs = pkt[slot, :, 0:d]
            r = pkt[slot, :, RC:RC + 1]
            acc = pkt[slot, :, ACC0:ACC1]
            for j in range(e_loc):
                w = (r == my_base + float(j)).astype(jnp.float32)
                acc = acc + jnp.dot(
                    xs * w, ew[j], preferred_element_type=jnp.float32)
            pkt[slot, :, ACC0:ACC1] = acc

        process(0)

        for hh in range(N_DEV):
            s_slot = hh % 2
            r_slot = (hh + 1) % 2
            if hh >= 1:
                pl.semaphore_wait(credit, 1)
            rdma = pltpu.make_async_remote_copy(
                src_ref=pkt.at[s_slot],
                dst_ref=pkt.at[r_slot],
                send_sem=send_sems.at[s_slot],
                recv_sem=recv_sems.at[r_slot],
                device_id=(right,),
                device_id_type=pl.DeviceIdType.MESH,
            )
            rdma.start()
            rdma.wait()
            if hh <= N_DEV - 2:
                pl.semaphore_signal(
                    credit, inc=1, device_id=(left,),
                    device_id_type=pl.DeviceIdType.MESH,
                )
            if hh < N_DEV - 1:
                process(r_slot)
            else:
                out_ref[...] = pkt[r_slot, :, ACC0:ACC1]

    return pl.pallas_call(
        body,
        out_shape=jax.ShapeDtypeStruct((n_tok, h), jnp.float32),
        in_specs=[pl.BlockSpec(memory_space=pltpu.VMEM)] * 5,
        out_specs=pl.BlockSpec(memory_space=pltpu.VMEM),
        scratch_shapes=[
            pltpu.VMEM((2, n_tok, PKT), jnp.float32),
            pltpu.SemaphoreType.DMA((2,)),
            pltpu.SemaphoreType.DMA((2,)),
            pltpu.SemaphoreType.REGULAR,
        ],
        compiler_params=pltpu.CompilerParams(collective_id=0),
    )(x, router_W, route_idx, expert_W, shared_W)


# device time: 323742 ns/iter; 1.2594x vs baseline; 1.2594x over previous
import jax
import jax.numpy as jnp
from jax import lax
from jax.experimental import pallas as pl
from jax.experimental.pallas import tpu as pltpu

N_DEV = 16


def kernel(x, router_W, route_idx, expert_W, shared_W):
    n_tok, d = x.shape
    n_exp = router_W.shape[1]
    e_loc = expert_W.shape[0]
    h = shared_W.shape[1]
    AW = d + 128

    def body(x_ref, rw_ref, idx_ref, ew_ref, sw_ref, out_ref,
             bufA, bufB, sendA_sems, recvA_sems, sendB_sems, recvB_sems,
             creditA, creditB):
        my = lax.axis_index("i")
        left = lax.rem(my + N_DEV - 1, N_DEV)
        right = lax.rem(my + 1, N_DEV)

        def descA(k):
            return pltpu.make_async_remote_copy(
                src_ref=bufA.at[k % 3],
                dst_ref=bufA.at[(k + 1) % 3],
                send_sem=sendA_sems.at[k % 3],
                recv_sem=recvA_sems.at[(k + 1) % 3],
                device_id=(right,),
                device_id_type=pl.DeviceIdType.MESH,
            )

        def descB(k):
            return pltpu.make_async_remote_copy(
                src_ref=bufB.at[k % 3],
                dst_ref=bufB.at[(k + 1) % 3],
                send_sem=sendB_sems.at[k % 3],
                recv_sem=recvB_sems.at[(k + 1) % 3],
                device_id=(right,),
                device_id_type=pl.DeviceIdType.MESH,
            )

        def sig(sem, dev):
            pl.semaphore_signal(
                sem, inc=1, device_id=(dev,),
                device_id_type=pl.DeviceIdType.MESH,
            )

        barrier = pltpu.get_barrier_semaphore()
        sig(barrier, left)
        sig(barrier, right)
        pl.semaphore_wait(barrier, 2)

        xv = x_ref[...]
        scores = jnp.dot(xv, rw_ref[...], preferred_element_type=jnp.float32)
        m = jnp.max(scores, axis=1, keepdims=True)
        p = jnp.exp(scores - m)
        p = p / jnp.sum(p, axis=1, keepdims=True)
        route = idx_ref[...]
        onehot = lax.broadcasted_iota(jnp.int32, (n_tok, n_exp), 1) == route
        coeff = jnp.sum(jnp.where(onehot, p, 0.0), axis=1, keepdims=True)
        bufA[0, :, 0:d] = xv * coeff
        bufA[0, :, d:AW] = jnp.broadcast_to(
            route.astype(jnp.float32), (n_tok, 128))

        ew = ew_ref[...]
        my_base = (my * e_loc).astype(jnp.float32)

        def contrib(slot):
            xs = bufA[slot, :, 0:d]
            rr = bufA[slot, :, d:d + 1]
            acc = jnp.zeros((n_tok, h), dtype=jnp.float32)
            for j in range(e_loc):
                w = (rr == my_base + float(j)).astype(jnp.float32)
                acc = acc + jnp.dot(
                    xs * w, ew[j], preferred_element_type=jnp.float32)
            return acc

        bufB[0, :, :] = jnp.dot(
            xv, sw_ref[...], preferred_element_type=jnp.float32) + contrib(0)

        descA(0).start()
        descB(0).start()

        for r in range(N_DEV - 1):
            ra = (r + 1) % 3
            descA(r).wait_recv()
            if r <= N_DEV - 3:
                if r >= 1:
                    pl.semaphore_wait(creditA, 1)
                descA(r + 1).start()
            C = contrib(ra)
            descA(r).wait_send()
            if r <= N_DEV - 4:
                sig(creditA, left)
            descB(r).wait_send()
            if r <= N_DEV - 3:
                sig(creditB, left)
            descB(r).wait_recv()
            bufB[ra, :, :] = bufB[ra, :, :] + C
            if r >= 1:
                pl.semaphore_wait(creditB, 1)
            descB(r + 1).start()

        descB(N_DEV - 1).wait_recv()
        out_ref[...] = bufB[N_DEV % 3, :, :]
        descB(N_DEV - 1).wait_send()

    return pl.pallas_call(
        body,
        out_shape=jax.ShapeDtypeStruct((n_tok, h), jnp.float32),
        in_specs=[pl.BlockSpec(memory_space=pltpu.VMEM)] * 5,
        out_specs=pl.BlockSpec(memory_space=pltpu.VMEM),
        scratch_shapes=[
            pltpu.VMEM((3, n_tok, AW), jnp.float32),
            pltpu.VMEM((3, n_tok, h), jnp.float32),
            pltpu.SemaphoreType.DMA((3,)),
            pltpu.SemaphoreType.DMA((3,)),
            pltpu.SemaphoreType.DMA((3,)),
            pltpu.SemaphoreType.DMA((3,)),
            pltpu.SemaphoreType.REGULAR,
            pltpu.SemaphoreType.REGULAR,
        ],
        compiler_params=pltpu.CompilerParams(collective_id=0),
    )(x, router_W, route_idx, expert_W, shared_W)


# device time: 170262 ns/iter; 2.3946x vs baseline; 1.9014x over previous
import jax
import jax.numpy as jnp
from jax import lax
from jax.experimental import pallas as pl
from jax.experimental.pallas import tpu as pltpu

N_DEV = 16


def kernel(x, router_W, route_idx, expert_W, shared_W):
    n_tok, d = x.shape
    n_exp = router_W.shape[1]
    e_loc = expert_W.shape[0]
    h = shared_W.shape[1]
    T = n_tok // 2
    AW = d + 128

    def body(x_ref, rw_ref, idx_ref, ew_ref, sw_ref, out_ref,
             bufA_cw, bufB_cw, bufA_ccw, bufB_ccw,
             sA_cw, rA_cw, sB_cw, rB_cw,
             sA_ccw, rA_ccw, sB_ccw, rB_ccw,
             crA_cw, crB_cw, crA_ccw, crB_ccw):
        my = lax.axis_index("i")
        left = lax.rem(my + N_DEV - 1, N_DEV)
        right = lax.rem(my + 1, N_DEV)

        def mkdesc(buf, ssem, rsem, dst):
            def desc(k):
                return pltpu.make_async_remote_copy(
                    src_ref=buf.at[k % 3],
                    dst_ref=buf.at[(k + 1) % 3],
                    send_sem=ssem.at[k % 3],
                    recv_sem=rsem.at[(k + 1) % 3],
                    device_id=(dst,),
                    device_id_type=pl.DeviceIdType.MESH,
                )
            return desc

        dAcw = mkdesc(bufA_cw, sA_cw, rA_cw, right)
        dBcw = mkdesc(bufB_cw, sB_cw, rB_cw, right)
        dAcc = mkdesc(bufA_ccw, sA_ccw, rA_ccw, left)
        dBcc = mkdesc(bufB_ccw, sB_ccw, rB_ccw, left)

        def sig(sem, dev):
            pl.semaphore_signal(
                sem, inc=1, device_id=(dev,),
                device_id_type=pl.DeviceIdType.MESH,
            )

        barrier = pltpu.get_barrier_semaphore()
        sig(barrier, left)
        sig(barrier, right)
        pl.semaphore_wait(barrier, 2)

        xv = x_ref[...]
        scores = jnp.dot(xv, rw_ref[...], preferred_element_type=jnp.float32)
        m = jnp.max(scores, axis=1, keepdims=True)
        p = jnp.exp(scores - m)
        p = p / jnp.sum(p, axis=1, keepdims=True)
        route = idx_ref[...]
        onehot = lax.broadcasted_iota(jnp.int32, (n_tok, n_exp), 1) == route
        coeff = jnp.sum(jnp.where(onehot, p, 0.0), axis=1, keepdims=True)
        xs = xv * coeff
        rbrd = jnp.broadcast_to(route.astype(jnp.float32), (n_tok, 128))
        bufA_cw[0, :, 0:d] = xs[0:T]
        bufA_cw[0, :, d:AW] = rbrd[0:T]
        bufA_ccw[0, :, 0:d] = xs[T:n_tok]
        bufA_ccw[0, :, d:AW] = rbrd[T:n_tok]

        ew = ew_ref[...]
        my_base = (my * e_loc).astype(jnp.float32)

        def contrib(buf, slot):
            xsl = buf[slot, :, 0:d]
            rr = buf[slot, :, d:d + 1]
            acc = jnp.zeros((T, h), dtype=jnp.float32)
            for j in range(e_loc):
                w = (rr == my_base + float(j)).astype(jnp.float32)
                acc = acc + jnp.dot(
                    xsl * w, ew[j], preferred_element_type=jnp.float32)
            return acc

        sh = jnp.dot(xv, sw_ref[...], preferred_element_type=jnp.float32)
        bufB_cw[0, :, :] = sh[0:T] + contrib(bufA_cw, 0)
        bufB_ccw[0, :, :] = sh[T:n_tok] + contrib(bufA_ccw, 0)

        dAcw(0).start()
        dAcc(0).start()
        dBcw(0).start()
        dBcc(0).start()

        for r in range(N_DEV - 1):
            ra = (r + 1) % 3
            dAcw(r).wait_recv()
            if r <= N_DEV - 3:
                if r >= 1:
                    pl.semaphore_wait(crA_cw, 1)
                dAcw(r + 1).start()
            dAcc(r).wait_recv()
            if r <= N_DEV - 3:
                if r >= 1:
                    pl.semaphore_wait(crA_ccw, 1)
                dAcc(r + 1).start()
            C_cw = contrib(bufA_cw, ra)
            C_cc = contrib(bufA_ccw, ra)
            dAcw(r).wait_send()
            if r <= N_DEV - 4:
                sig(crA_cw, left)
            dAcc(r).wait_send()
            if r <= N_DEV - 4:
                sig(crA_ccw, right)
            dBcw(r).wait_send()
            if r <= N_DEV - 3:
                sig(crB_cw, left)
            dBcc(r).wait_send()
            if r <= N_DEV - 3:
                sig(crB_ccw, right)
            dBcw(r).wait_recv()
            bufB_cw[ra, :, :] = bufB_cw[ra, :, :] + C_cw
            if r >= 1:
                pl.semaphore_wait(crB_cw, 1)
            dBcw(r + 1).start()
            dBcc(r).wait_recv()
            bufB_ccw[ra, :, :] = bufB_ccw[ra, :, :] + C_cc
            if r >= 1:
                pl.semaphore_wait(crB_ccw, 1)
            dBcc(r + 1).start()

        fs = N_DEV % 3
        dBcw(N_DEV - 1).wait_recv()
        out_ref[0:T, :] = bufB_cw[fs, :, :]
        dBcc(N_DEV - 1).wait_recv()
        out_ref[T:n_tok, :] = bufB_ccw[fs, :, :]
        dBcw(N_DEV - 1).wait_send()
        dBcc(N_DEV - 1).wait_send()

    return pl.pallas_call(
        body,
        out_shape=jax.ShapeDtypeStruct((n_tok, h), jnp.float32),
        in_specs=[pl.BlockSpec(memory_space=pltpu.VMEM)] * 5,
        out_specs=pl.BlockSpec(memory_space=pltpu.VMEM),
        scratch_shapes=[
            pltpu.VMEM((3, T, AW), jnp.float32),
            pltpu.VMEM((3, T, h), jnp.float32),
            pltpu.VMEM((3, T, AW), jnp.float32),
            pltpu.VMEM((3, T, h), jnp.float32),
            pltpu.SemaphoreType.DMA((3,)),
            pltpu.SemaphoreType.DMA((3,)),
            pltpu.SemaphoreType.DMA((3,)),
            pltpu.SemaphoreType.DMA((3,)),
            pltpu.SemaphoreType.DMA((3,)),
            pltpu.SemaphoreType.DMA((3,)),
            pltpu.SemaphoreType.DMA((3,)),
            pltpu.SemaphoreType.DMA((3,)),
            pltpu.SemaphoreType.REGULAR,
            pltpu.SemaphoreType.REGULAR,
            pltpu.SemaphoreType.REGULAR,
            pltpu.SemaphoreType.REGULAR,
        ],
        compiler_params=pltpu.CompilerParams(collective_id=0),
    )(x, router_W, route_idx, expert_W, shared_W)


# device time: 97662 ns/iter; 4.1747x vs baseline; 1.7434x over previous
import jax
import jax.numpy as jnp
from jax import lax
from jax.experimental import pallas as pl
from jax.experimental.pallas import tpu as pltpu

N_DEV = 16


def kernel(x, router_W, route_idx, expert_W, shared_W):
    n_tok, d = x.shape
    n_exp = router_W.shape[1]
    e_loc = expert_W.shape[0]
    h = shared_W.shape[1]
    T = n_tok // 2
    AW = d + 128

    def body(x_ref, rw_ref, idx_ref, ew_ref, sw_ref, out_ref,
             bufA_cw, bufB_cw, bufA_ccw, bufB_ccw,
             sA_cw, rA_cw, sB_cw, rB_cw,
             sA_ccw, rA_ccw, sB_ccw, rB_ccw,
             crA_cw, crB_cw, crA_ccw, crB_ccw):
        my = lax.axis_index("i")
        left = lax.rem(my + N_DEV - 1, N_DEV)
        right = lax.rem(my + 1, N_DEV)

        def mkdesc(buf, ssem, rsem, dst):
            def desc(k):
                return pltpu.make_async_remote_copy(
                    src_ref=buf.at[k % 3],
                    dst_ref=buf.at[(k + 1) % 3],
                    send_sem=ssem.at[k % 3],
                    recv_sem=rsem.at[(k + 1) % 3],
                    device_id=(dst,),
                    device_id_type=pl.DeviceIdType.MESH,
                )
            return desc

        dAcw = mkdesc(bufA_cw, sA_cw, rA_cw, right)
        dBcw = mkdesc(bufB_cw, sB_cw, rB_cw, right)
        dAcc = mkdesc(bufA_ccw, sA_ccw, rA_ccw, left)
        dBcc = mkdesc(bufB_ccw, sB_ccw, rB_ccw, left)

        def sig(sem, dev):
            pl.semaphore_signal(
                sem, inc=1, device_id=(dev,),
                device_id_type=pl.DeviceIdType.MESH,
            )

        barrier = pltpu.get_barrier_semaphore()
        sig(barrier, left)
        sig(barrier, right)
        pl.semaphore_wait(barrier, 2)

        xv = x_ref[...]
        scores = jnp.dot(xv, rw_ref[...], preferred_element_type=jnp.float32)
        m = jnp.max(scores, axis=1, keepdims=True)
        p = jnp.exp(scores - m)
        p = p / jnp.sum(p, axis=1, keepdims=True)
        route = idx_ref[...]
        onehot = lax.broadcasted_iota(jnp.int32, (n_tok, n_exp), 1) == route
        coeff = jnp.sum(jnp.where(onehot, p, 0.0), axis=1, keepdims=True)
        xs = (xv * coeff).astype(jnp.bfloat16)
        rbrd = jnp.broadcast_to(
            route.astype(jnp.bfloat16), (n_tok, 128))
        bufA_cw[0, :, 0:d] = xs[0:T]
        bufA_cw[0, :, d:AW] = rbrd[0:T]
        bufA_ccw[0, :, 0:d] = xs[T:n_tok]
        bufA_ccw[0, :, d:AW] = rbrd[T:n_tok]

        ew = ew_ref[...].astype(jnp.bfloat16)
        my_base = (my * e_loc).astype(jnp.float32)

        def contrib(buf, slot):
            xsl = buf[slot, :, 0:d]
            rr = buf[slot, :, d:d + 1].astype(jnp.float32)
            acc = jnp.zeros((T, h), dtype=jnp.float32)
            for j in range(e_loc):
                w = (rr == my_base + float(j)).astype(jnp.bfloat16)
                acc = acc + jnp.dot(
                    xsl * w, ew[j], preferred_element_type=jnp.float32)
            return acc

        sh = jnp.dot(xv, sw_ref[...], preferred_element_type=jnp.float32)
        bufB_cw[0, :, :] = (sh[0:T] + contrib(bufA_cw, 0)).astype(jnp.bfloat16)
        bufB_ccw[0, :, :] = (
            sh[T:n_tok] + contrib(bufA_ccw, 0)).astype(jnp.bfloat16)

        dAcw(0).start()
        dAcc(0).start()
        dBcw(0).start()
        dBcc(0).start()

        for r in range(N_DEV - 1):
            ra = (r + 1) % 3
            dAcw(r).wait_recv()
            if r <= N_DEV - 3:
                if r >= 1:
                    pl.semaphore_wait(crA_cw, 1)
                dAcw(r + 1).start()
            dAcc(r).wait_recv()
            if r <= N_DEV - 3:
                if r >= 1:
                    pl.semaphore_wait(crA_ccw, 1)
                dAcc(r + 1).start()
            C_cw = contrib(bufA_cw, ra)
            C_cc = contrib(bufA_ccw, ra)
            dAcw(r).wait_send()
            if r <= N_DEV - 4:
                sig(crA_cw, left)
            dAcc(r).wait_send()
            if r <= N_DEV - 4:
                sig(crA_ccw, right)
            dBcw(r).wait_send()
            if r <= N_DEV - 3:
                sig(crB_cw, left)
            dBcc(r).wait_send()
            if r <= N_DEV - 3:
                sig(crB_ccw, right)
            dBcw(r).wait_recv()
            bufB_cw[ra, :, :] = (
                bufB_cw[ra, :, :] + C_cw).astype(jnp.bfloat16)
            if r >= 1:
                pl.semaphore_wait(crB_cw, 1)
            dBcw(r + 1).start()
            dBcc(r).wait_recv()
            bufB_ccw[ra, :, :] = (
                bufB_ccw[ra, :, :] + C_cc).astype(jnp.bfloat16)
            if r >= 1:
                pl.semaphore_wait(crB_ccw, 1)
            dBcc(r + 1).start()

        fs = N_DEV % 3
        dBcw(N_DEV - 1).wait_recv()
        out_ref[0:T, :] = bufB_cw[fs, :, :].astype(jnp.float32)
        dBcc(N_DEV - 1).wait_recv()
        out_ref[T:n_tok, :] = bufB_ccw[fs, :, :].astype(jnp.float32)
        dBcw(N_DEV - 1).wait_send()
        dBcc(N_DEV - 1).wait_send()

    return pl.pallas_call(
        body,
        out_shape=jax.ShapeDtypeStruct((n_tok, h), jnp.float32),
        in_specs=[pl.BlockSpec(memory_space=pltpu.VMEM)] * 5,
        out_specs=pl.BlockSpec(memory_space=pltpu.VMEM),
        scratch_shapes=[
            pltpu.VMEM((3, T, AW), jnp.bfloat16),
            pltpu.VMEM((3, T, h), jnp.bfloat16),
            pltpu.VMEM((3, T, AW), jnp.bfloat16),
            pltpu.VMEM((3, T, h), jnp.bfloat16),
            pltpu.SemaphoreType.DMA((3,)),
            pltpu.SemaphoreType.DMA((3,)),
            pltpu.SemaphoreType.DMA((3,)),
            pltpu.SemaphoreType.DMA((3,)),
            pltpu.SemaphoreType.DMA((3,)),
            pltpu.SemaphoreType.DMA((3,)),
            pltpu.SemaphoreType.DMA((3,)),
            pltpu.SemaphoreType.DMA((3,)),
            pltpu.SemaphoreType.REGULAR,
            pltpu.SemaphoreType.REGULAR,
            pltpu.SemaphoreType.REGULAR,
            pltpu.SemaphoreType.REGULAR,
        ],
        compiler_params=pltpu.CompilerParams(collective_id=0),
    )(x, router_W, route_idx, expert_W, shared_W)


# device time: 95322 ns/iter; 4.2772x vs baseline; 1.0245x over previous
import jax
import jax.numpy as jnp
from jax import lax
from jax.experimental import pallas as pl
from jax.experimental.pallas import tpu as pltpu

N_DEV = 16


def kernel(x, router_W, route_idx, expert_W, shared_W):
    n_tok, d = x.shape
    n_exp = router_W.shape[1]
    e_loc = expert_W.shape[0]
    h = shared_W.shape[1]
    T = n_tok // 2
    AW = d + 128

    def body(x_ref, rw_ref, idx_ref, ew_ref, sw_ref, out_ref,
             bufA_cw, bufB_cw, bufA_ccw, bufB_ccw,
             sA_cw, rA_cw, sB_cw, rB_cw,
             sA_ccw, rA_ccw, sB_ccw, rB_ccw,
             crA_cw, crB_cw, crA_ccw, crB_ccw):
        my = lax.axis_index("i")
        left = lax.rem(my + N_DEV - 1, N_DEV)
        right = lax.rem(my + 1, N_DEV)

        def mkdesc(buf, ssem, rsem, dst):
            def desc(k):
                return pltpu.make_async_remote_copy(
                    src_ref=buf.at[k % 3],
                    dst_ref=buf.at[(k + 1) % 3],
                    send_sem=ssem.at[k % 3],
                    recv_sem=rsem.at[(k + 1) % 3],
                    device_id=(dst,),
                    device_id_type=pl.DeviceIdType.MESH,
                )
            return desc

        dAcw = mkdesc(bufA_cw, sA_cw, rA_cw, right)
        dBcw = mkdesc(bufB_cw, sB_cw, rB_cw, right)
        dAcc = mkdesc(bufA_ccw, sA_ccw, rA_ccw, left)
        dBcc = mkdesc(bufB_ccw, sB_ccw, rB_ccw, left)

        def sig(sem, dev):
            pl.semaphore_signal(
                sem, inc=1, device_id=(dev,),
                device_id_type=pl.DeviceIdType.MESH,
            )

        barrier = pltpu.get_barrier_semaphore()
        sig(barrier, left)
        sig(barrier, right)
        pl.semaphore_wait(barrier, 2)

        xv = x_ref[...]
        scores = jnp.dot(xv, rw_ref[...], preferred_element_type=jnp.float32)
        m = jnp.max(scores, axis=1, keepdims=True)
        p = jnp.exp(scores - m)
        p = p / jnp.sum(p, axis=1, keepdims=True)
        route = idx_ref[...]
        onehot = lax.broadcasted_iota(jnp.int32, (n_tok, n_exp), 1) == route
        coeff = jnp.sum(jnp.where(onehot, p, 0.0), axis=1, keepdims=True)
        xs = (xv * coeff).astype(jnp.bfloat16)
        rbrd = jnp.broadcast_to(
            route.astype(jnp.bfloat16), (n_tok, 128))
        bufA_cw[0, :, 0:d] = xs[0:T]
        bufA_cw[0, :, d:AW] = rbrd[0:T]
        bufA_ccw[0, :, 0:d] = xs[T:n_tok]
        bufA_ccw[0, :, d:AW] = rbrd[T:n_tok]

        ew = ew_ref[...].astype(jnp.bfloat16)
        my_base = (my * e_loc).astype(jnp.float32)

        def contrib(buf, slot):
            xsl = buf[slot, :, 0:d]
            rr = buf[slot, :, d:d + 1].astype(jnp.float32)
            acc = jnp.zeros((T, h), dtype=jnp.float32)
            for j in range(e_loc):
                w = (rr == my_base + float(j)).astype(jnp.bfloat16)
                acc = acc + jnp.dot(
                    xsl * w, ew[j], preferred_element_type=jnp.float32)
            return acc

        sh = jnp.dot(xv, sw_ref[...], preferred_element_type=jnp.float32)
        home_cw = sh[0:T] + contrib(bufA_cw, 0)
        home_cc = sh[T:n_tok] + contrib(bufA_ccw, 0)

        dAcw(0).start()
        dAcc(0).start()

        for r in range(N_DEV - 1):
            ra = (r + 1) % 3
            dAcw(r).wait_recv()
            if r <= N_DEV - 3:
                if r >= 1:
                    pl.semaphore_wait(crA_cw, 1)
                dAcw(r + 1).start()
            dAcc(r).wait_recv()
            if r <= N_DEV - 3:
                if r >= 1:
                    pl.semaphore_wait(crA_ccw, 1)
                dAcc(r + 1).start()
            C_cw = contrib(bufA_cw, ra)
            C_cc = contrib(bufA_ccw, ra)
            dAcw(r).wait_send()
            if r <= N_DEV - 4:
                sig(crA_cw, left)
            dAcc(r).wait_send()
            if r <= N_DEV - 4:
                sig(crA_ccw, right)
            if r == 0:
                bufB_cw[ra, :, :] = C_cw.astype(jnp.bfloat16)
            else:
                dBcw(r).wait_send()
                if r <= N_DEV - 3:
                    sig(crB_cw, left)
                dBcw(r).wait_recv()
                bufB_cw[ra, :, :] = (
                    bufB_cw[ra, :, :] + C_cw).astype(jnp.bfloat16)
                if r >= 2:
                    pl.semaphore_wait(crB_cw, 1)
            dBcw(r + 1).start()
            if r == 0:
                bufB_ccw[ra, :, :] = C_cc.astype(jnp.bfloat16)
            else:
                dBcc(r).wait_send()
                if r <= N_DEV - 3:
                    sig(crB_ccw, right)
                dBcc(r).wait_recv()
                bufB_ccw[ra, :, :] = (
                    bufB_ccw[ra, :, :] + C_cc).astype(jnp.bfloat16)
                if r >= 2:
                    pl.semaphore_wait(crB_ccw, 1)
            dBcc(r + 1).start()

        fs = N_DEV % 3
        dBcw(N_DEV - 1).wait_recv()
        out_ref[0:T, :] = bufB_cw[fs, :, :] + home_cw
        dBcc(N_DEV - 1).wait_recv()
        out_ref[T:n_tok, :] = bufB_ccw[fs, :, :] + home_cc
        dBcw(N_DEV - 1).wait_send()
        dBcc(N_DEV - 1).wait_send()

    return pl.pallas_call(
        body,
        out_shape=jax.ShapeDtypeStruct((n_tok, h), jnp.float32),
        in_specs=[pl.BlockSpec(memory_space=pltpu.VMEM)] * 5,
        out_specs=pl.BlockSpec(memory_space=pltpu.VMEM),
        scratch_shapes=[
            pltpu.VMEM((3, T, AW), jnp.bfloat16),
            pltpu.VMEM((3, T, h), jnp.bfloat16),
            pltpu.VMEM((3, T, AW), jnp.bfloat16),
            pltpu.VMEM((3, T, h), jnp.bfloat16),
            pltpu.SemaphoreType.DMA((3,)),
            pltpu.SemaphoreType.DMA((3,)),
            pltpu.SemaphoreType.DMA((3,)),
            pltpu.SemaphoreType.DMA((3,)),
            pltpu.SemaphoreType.DMA((3,)),
            pltpu.SemaphoreType.DMA((3,)),
            pltpu.SemaphoreType.DMA((3,)),
            pltpu.SemaphoreType.DMA((3,)),
            pltpu.SemaphoreType.REGULAR,
            pltpu.SemaphoreType.REGULAR,
            pltpu.SemaphoreType.REGULAR,
            pltpu.SemaphoreType.REGULAR,
        ],
        compiler_params=pltpu.CompilerParams(collective_id=0),
    )(x, router_W, route_idx, expert_W, shared_W)
